# baseline (speedup 1.0000x reference)
import jax
import jax.numpy as jnp
from jax.experimental import pallas as pl
from jax.experimental.pallas import tpu as pltpu

D_MODEL = 1024
D_FF = 2816
NUM_EXPERTS = 16
N_TOK = 32
F_BLK = 1408
NF = D_FF // F_BLK


def _moe_kernel(x_ref, wr_ref, w1_ref, w3_ref, w2_ref, out_ref,
                xT_ref, wT_ref, h_ref, outT_ref):
    e = pl.program_id(0)
    f = pl.program_id(1)

    @pl.when(jnp.logical_and(e == 0, f == 0))
    def _prologue():
        xT = x_ref[...].T
        xT_ref[...] = xT
        lT = jnp.dot(wr_ref[...], xT,
                     preferred_element_type=jnp.float32)
        m = jnp.max(lT, axis=0, keepdims=True)
        ex = jnp.exp(lT - m)
        p = ex / jnp.sum(ex, axis=0, keepdims=True)
        iota_e = jax.lax.broadcasted_iota(jnp.int32, (NUM_EXPERTS, N_TOK), 0)
        m1 = jnp.max(p, axis=0, keepdims=True)
        i1 = jnp.min(jnp.where(p == m1, iota_e, NUM_EXPERTS),
                     axis=0, keepdims=True)
        first = iota_e == i1
        pm = jnp.where(first, -1.0, p)
        m2 = jnp.max(pm, axis=0, keepdims=True)
        i2 = jnp.min(jnp.where(pm == m2, iota_e, NUM_EXPERTS),
                     axis=0, keepdims=True)
        second = iota_e == i2
        keep = jnp.logical_or(first, second)
        denom = m1 + m2 + 1e-9
        wT_ref[...] = jnp.where(keep, p, 0.0) / denom
        outT_ref[...] = jnp.zeros_like(outT_ref)

    xT = xT_ref[...]
    h1 = jnp.dot(w1_ref[0], xT, preferred_element_type=jnp.float32)
    h3 = jnp.dot(w3_ref[0], xT, preferred_element_type=jnp.float32)
    sel = jax.lax.broadcasted_iota(jnp.int32, (NUM_EXPERTS, 1), 0) == e
    wrow = jnp.sum(jnp.where(sel, wT_ref[...], 0.0),
                   axis=0, keepdims=True)
    h_ref[pl.ds(f * F_BLK, F_BLK), :] = (h1 * jax.nn.sigmoid(h1)) * h3 * wrow

    @pl.when(f == NF - 1)
    def _expert_out():
        outT_ref[...] += jnp.dot(w2_ref[0], h_ref[...],
                                 preferred_element_type=jnp.float32)

    @pl.when(jnp.logical_and(e == NUM_EXPERTS - 1, f == NF - 1))
    def _epilogue():
        out_ref[...] = outT_ref[...].T


def kernel(x, W_router, W1, W3, W2):
    b, s, d = x.shape
    n = b * s
    out = pl.pallas_call(
        _moe_kernel,
        grid=(NUM_EXPERTS, NF),
        in_specs=[
            pl.BlockSpec((n, d), lambda e, f: (0, 0)),
            pl.BlockSpec((NUM_EXPERTS, d), lambda e, f: (0, 0)),
            pl.BlockSpec((1, F_BLK, d), lambda e, f: (e, f, 0)),
            pl.BlockSpec((1, F_BLK, d), lambda e, f: (e, f, 0)),
            pl.BlockSpec((1, d, D_FF), lambda e, f: (e, 0, 0)),
        ],
        out_specs=pl.BlockSpec((n, d), lambda e, f: (0, 0)),
        out_shape=jax.ShapeDtypeStruct((n, d), jnp.float32),
        scratch_shapes=[
            pltpu.VMEM((D_MODEL, N_TOK), jnp.float32),
            pltpu.VMEM((NUM_EXPERTS, N_TOK), jnp.float32),
            pltpu.VMEM((D_FF, N_TOK), jnp.float32),
            pltpu.VMEM((D_MODEL, N_TOK), jnp.float32),
        ],
    )(x.reshape(n, d), W_router, W1, W3, W2)
    return out.reshape(b, s, d)


# speedup vs baseline: 1.0008x; 1.0008x over previous
import jax
import jax.numpy as jnp
from jax.experimental import pallas as pl
from jax.experimental.pallas import tpu as pltpu

D_MODEL = 1024
D_FF = 2816
NUM_EXPERTS = 16
N_TOK = 32
F_BLK = 1408
NF = D_FF // F_BLK


def _moe_kernel(x_ref, wr_ref, w1_ref, w3_ref, w2_ref, out_ref,
                xT_ref, wT_ref, outT_ref):
    e = pl.program_id(0)
    f = pl.program_id(1)

    @pl.when(jnp.logical_and(e == 0, f == 0))
    def _prologue():
        xT = x_ref[...].T
        xT_ref[...] = xT
        lT = jnp.dot(wr_ref[...], xT,
                     preferred_element_type=jnp.float32)
        m = jnp.max(lT, axis=0, keepdims=True)
        ex = jnp.exp(lT - m)
        p = ex / jnp.sum(ex, axis=0, keepdims=True)
        iota_e = jax.lax.broadcasted_iota(jnp.int32, (NUM_EXPERTS, N_TOK), 0)
        m1 = jnp.max(p, axis=0, keepdims=True)
        i1 = jnp.min(jnp.where(p == m1, iota_e, NUM_EXPERTS),
                     axis=0, keepdims=True)
        first = iota_e == i1
        pm = jnp.where(first, -1.0, p)
        m2 = jnp.max(pm, axis=0, keepdims=True)
        i2 = jnp.min(jnp.where(pm == m2, iota_e, NUM_EXPERTS),
                     axis=0, keepdims=True)
        second = iota_e == i2
        keep = jnp.logical_or(first, second)
        denom = m1 + m2 + 1e-9
        wT_ref[...] = jnp.where(keep, p, 0.0) / denom
        outT_ref[...] = jnp.zeros_like(outT_ref)

    xT = xT_ref[...]
    h1 = jnp.dot(w1_ref[0], xT, preferred_element_type=jnp.float32)
    h3 = jnp.dot(w3_ref[0], xT, preferred_element_type=jnp.float32)
    sel = jax.lax.broadcasted_iota(jnp.int32, (NUM_EXPERTS, 1), 0) == e
    wrow = jnp.sum(jnp.where(sel, wT_ref[...], 0.0),
                   axis=0, keepdims=True)
    h = (h1 * jax.nn.sigmoid(h1)) * h3 * wrow
    outT_ref[...] += jnp.dot(w2_ref[0], h,
                             preferred_element_type=jnp.float32)

    @pl.when(jnp.logical_and(e == NUM_EXPERTS - 1, f == NF - 1))
    def _epilogue():
        out_ref[...] = outT_ref[...].T


def kernel(x, W_router, W1, W3, W2):
    b, s, d = x.shape
    n = b * s
    out = pl.pallas_call(
        _moe_kernel,
        grid=(NUM_EXPERTS, NF),
        in_specs=[
            pl.BlockSpec((n, d), lambda e, f: (0, 0)),
            pl.BlockSpec((NUM_EXPERTS, d), lambda e, f: (0, 0)),
            pl.BlockSpec((1, F_BLK, d), lambda e, f: (e, f, 0)),
            pl.BlockSpec((1, F_BLK, d), lambda e, f: (e, f, 0)),
            pl.BlockSpec((1, d, F_BLK), lambda e, f: (e, 0, f)),
        ],
        out_specs=pl.BlockSpec((n, d), lambda e, f: (0, 0)),
        out_shape=jax.ShapeDtypeStruct((n, d), jnp.float32),
        scratch_shapes=[
            pltpu.VMEM((D_MODEL, N_TOK), jnp.float32),
            pltpu.VMEM((NUM_EXPERTS, N_TOK), jnp.float32),
            pltpu.VMEM((D_MODEL, N_TOK), jnp.float32),
        ],
    )(x.reshape(n, d), W_router, W1, W3, W2)
    return out.reshape(b, s, d)


# speedup vs baseline: 1.0476x; 1.0468x over previous
import jax
import jax.numpy as jnp
from jax.experimental import pallas as pl
from jax.experimental.pallas import tpu as pltpu

D_MODEL = 1024
D_FF = 2816
NUM_EXPERTS = 16
N_TOK = 32
F_BLK = 1408
NF = D_FF // F_BLK


def _moe_kernel(xT_ref, wr_ref, w1_ref, w3_ref, w2_ref, out_ref, wT_ref):
    e = pl.program_id(0)
    f = pl.program_id(1)

    @pl.when(jnp.logical_and(e == 0, f == 0))
    def _router():
        xT = xT_ref[...]
        lT = jnp.dot(wr_ref[...], xT,
                     preferred_element_type=jnp.float32)
        m = jnp.max(lT, axis=0, keepdims=True)
        ex = jnp.exp(lT - m)
        p = ex / jnp.sum(ex, axis=0, keepdims=True)
        iota_e = jax.lax.broadcasted_iota(jnp.int32, (NUM_EXPERTS, N_TOK), 0)
        m1 = jnp.max(p, axis=0, keepdims=True)
        i1 = jnp.min(jnp.where(p == m1, iota_e, NUM_EXPERTS),
                     axis=0, keepdims=True)
        first = iota_e == i1
        pm = jnp.where(first, -1.0, p)
        m2 = jnp.max(pm, axis=0, keepdims=True)
        i2 = jnp.min(jnp.where(pm == m2, iota_e, NUM_EXPERTS),
                     axis=0, keepdims=True)
        second = iota_e == i2
        keep = jnp.logical_or(first, second)
        denom = m1 + m2 + 1e-9
        wT_ref[...] = jnp.where(keep, p, 0.0) / denom
        out_ref[...] = jnp.zeros_like(out_ref)

    xT = xT_ref[...]
    h1 = jnp.dot(w1_ref[0], xT, preferred_element_type=jnp.float32)
    h3 = jnp.dot(w3_ref[0], xT, preferred_element_type=jnp.float32)
    h = (h1 * jax.nn.sigmoid(h1)) * h3
    sel = jax.lax.broadcasted_iota(jnp.int32, (NUM_EXPERTS, 1), 0) == e
    wrow = jnp.sum(jnp.where(sel, wT_ref[...], 0.0),
                   axis=0, keepdims=True)
    out_ref[...] += jnp.dot(w2_ref[0], h * wrow,
                            preferred_element_type=jnp.float32)


def kernel(x, W_router, W1, W3, W2):
    b, s, d = x.shape
    n = b * s
    xT = x.reshape(n, d).T
    out_t = pl.pallas_call(
        _moe_kernel,
        grid=(NUM_EXPERTS, NF),
        in_specs=[
            pl.BlockSpec((d, n), lambda e, f: (0, 0)),
            pl.BlockSpec((NUM_EXPERTS, d), lambda e, f: (0, 0)),
            pl.BlockSpec((1, F_BLK, d), lambda e, f: (e, f, 0)),
            pl.BlockSpec((1, F_BLK, d), lambda e, f: (e, f, 0)),
            pl.BlockSpec((1, d, F_BLK), lambda e, f: (e, 0, f)),
        ],
        out_specs=pl.BlockSpec((d, n), lambda e, f: (0, 0)),
        out_shape=jax.ShapeDtypeStruct((d, n), jnp.float32),
        scratch_shapes=[pltpu.VMEM((NUM_EXPERTS, n), jnp.float32)],
    )(xT, W_router, W1, W3, W2)
    return out_t.T.reshape(b, s, d)
